# baseline (speedup 1.0000x reference)
import functools

import jax
import jax.numpy as jnp
from jax import lax
from jax.experimental import pallas as pl
from jax.experimental.pallas import tpu as pltpu
from jax.experimental.pallas import tpu_sc as plsc

N = 10000
E = 320000
F2 = 128
B = 64
A = 256

CW = 125
EPAD = E
NROWS = EPAD // CW
NWORK = 32
ROWS_W = NROWS // NWORK
NPAD = 10240
NSEG = NPAD // 16
CNTW = 128
TS = 16


def _leaky(v):
    return jnp.where(v >= 0, v, 0.01 * v)


def _dot_t(a, w):
    return lax.dot_general(a, w, (((1,), (1,)), ((), ())),
                           preferred_element_type=jnp.float32)


def _sc_segsum_body(h_hbm, src_hbm, dst_hbm, zrow_hbm, agg_hbm,
                    src_v, dst_v, b0, b1, acc_sh, g0, g1, s0, s1):
    c = lax.axis_index("c")
    s = lax.axis_index("s")
    base = (c * 16 + s) * ROWS_W
    bufs = (b0, b1)
    gsem = (g0, g1)
    ssem = (s0, s1)

    pltpu.sync_copy(zrow_hbm, acc_sh.at[pl.ds(s * NSEG, NSEG)])
    plsc.subcore_barrier()

    def gat(q, j):
        pltpu.async_copy(h_hbm.at[src_v.at[q]], bufs[j], gsem[j])

    def wg(j):
        pltpu.make_async_copy(h_hbm.at[src_v.at[0]], bufs[j], gsem[j]).wait()

    def sca(q, j):
        pltpu.async_copy(bufs[j], acc_sh.at[dst_v.at[q]], ssem[j], add=True)

    def ws(j):
        pltpu.make_async_copy(bufs[j], acc_sh.at[dst_v.at[0]],
                              ssem[j]).wait()

    for t in range(ROWS_W // TS):
        if t > 0:
            wg(0)
            wg(1)
        pltpu.sync_copy(src_hbm.at[pl.ds(base + t * TS, TS)], src_v)
        pltpu.sync_copy(dst_hbm.at[pl.ds(base + t * TS, TS)], dst_v)
        gat(0, 0)
        gat(1, 1)

        def step(k, _):
            q0 = 2 * k
            wg(0)
            sca(q0, 0)
            wg(1)
            sca(q0 + 1, 1)
            ws(0)
            gat((q0 + 2) % TS, 0)
            ws(1)
            gat((q0 + 3) % TS, 1)
            return 0

        lax.fori_loop(0, TS // 2, step, 0)

    wg(0)
    wg(1)

    plsc.subcore_barrier()
    sl = pl.ds(s * NSEG, NSEG)
    pltpu.sync_copy(acc_sh.at[sl], agg_hbm.at[c].at[sl])


@functools.cache
def _sc_segsum():
    return pl.kernel(
        _sc_segsum_body,
        out_type=jax.ShapeDtypeStruct((2, NPAD, F2), jnp.float32),
        mesh=plsc.VectorSubcoreMesh(core_axis_name="c", subcore_axis_name="s",
                                    num_cores=2, num_subcores=16),
        scratch_types=[
            pltpu.VMEM((TS, CW), jnp.int32),
            pltpu.VMEM((TS, CW), jnp.int32),
            pltpu.VMEM((CW, F2), jnp.float32),
            pltpu.VMEM((CW, F2), jnp.float32),
            pltpu.VMEM_SHARED((NPAD, F2), jnp.float32),
            pltpu.SemaphoreType.DMA,
            pltpu.SemaphoreType.DMA,
            pltpu.SemaphoreType.DMA,
            pltpu.SemaphoreType.DMA,
        ],
    )


def _sc_cnt_body(dst_hbm, zcnt_hbm, ones_hbm, cnt_hbm,
                 dst_v, ones_v, cnt_sh):
    c = lax.axis_index("c")
    s = lax.axis_index("s")
    base = (c * 16 + s) * ROWS_W

    pltpu.sync_copy(ones_hbm, ones_v)
    pltpu.sync_copy(zcnt_hbm, cnt_sh.at[pl.ds(s * NSEG, NSEG)])
    plsc.subcore_barrier()

    for t in range(ROWS_W // TS):
        pltpu.sync_copy(dst_hbm.at[pl.ds(base + t * TS, TS)], dst_v)

        def step(q, _):
            pltpu.sync_copy(ones_v, cnt_sh.at[dst_v.at[q]], add=True)
            return 0

        lax.fori_loop(0, TS, step, 0)

    plsc.subcore_barrier()
    sl = pl.ds(s * NSEG, NSEG)
    pltpu.sync_copy(cnt_sh.at[sl], cnt_hbm.at[c].at[sl])


@functools.cache
def _sc_cnt():
    return pl.kernel(
        _sc_cnt_body,
        out_type=jax.ShapeDtypeStruct((2, NPAD, CNTW), jnp.float32),
        mesh=plsc.VectorSubcoreMesh(core_axis_name="c", subcore_axis_name="s",
                                    num_cores=2, num_subcores=16),
        scratch_types=[
            pltpu.VMEM((TS, CW), jnp.int32),
            pltpu.VMEM((CW, CNTW), jnp.float32),
            pltpu.VMEM_SHARED((NPAD, CNTW), jnp.float32),
        ],
    )


def _seg_sum(h, src2, dst2, consts):
    zrow, _, _ = consts
    return _sc_segsum()(h, src2, dst2, zrow)


BN = 1000
NB = N // BN


def _embed_body(x_ref, emb_ref, out_ref):
    bn = out_ref.shape[0]
    lanes = lax.broadcasted_iota(jnp.int32, (bn, F2), 1)
    oh0 = (lanes == x_ref[0, 0, :][:, None]).astype(jnp.float32)
    oh1 = (lanes == x_ref[0, 1, :][:, None]).astype(jnp.float32)
    h0 = jnp.dot(oh0, emb_ref[...], preferred_element_type=jnp.float32)
    h1 = jnp.dot(oh1, emb_ref[...], preferred_element_type=jnp.float32)
    out_ref[...] = jnp.concatenate([h0, h1], axis=1)


def _embed(xb, emb):
    return pl.pallas_call(
        _embed_body,
        grid=(NB,),
        in_specs=[
            pl.BlockSpec((1, 2, BN), lambda j: (j, 0, 0)),
            pl.BlockSpec((F2, 64), lambda j: (0, 0)),
        ],
        out_specs=pl.BlockSpec((BN, F2), lambda j: (j, 0)),
        out_shape=jax.ShapeDtypeStruct((N, F2), jnp.float32),
    )(xb, emb)


def _sage_body(aggp_ref, cntp_ref, h_ref, wl_ref, wr_ref, bs_ref, wg_ref,
               out_ref):
    agg = aggp_ref[0] + aggp_ref[1]
    cnt = cntp_ref[0, :, 0] + cntp_ref[1, :, 0]
    mean = agg / jnp.maximum(cnt, 1.0)[:, None]
    h = h_ref[...]
    sa = _leaky(_dot_t(mean, wl_ref[...]) + _dot_t(h, wr_ref[...])
                + bs_ref[...])
    out_ref[...] = _dot_t(sa, wg_ref[...])


def _sage_gcn_pre(aggp, cntp, h, wl, wr, bs, wg):
    return pl.pallas_call(
        _sage_body,
        grid=(NB,),
        in_specs=[
            pl.BlockSpec((2, BN, F2), lambda j: (0, j, 0)),
            pl.BlockSpec((2, BN, CNTW), lambda j: (0, j, 0)),
            pl.BlockSpec((BN, F2), lambda j: (j, 0)),
            pl.BlockSpec((F2, F2), lambda j: (0, 0)),
            pl.BlockSpec((F2, F2), lambda j: (0, 0)),
            pl.BlockSpec((1, F2), lambda j: (0, 0)),
            pl.BlockSpec((F2, F2), lambda j: (0, 0)),
        ],
        out_specs=pl.BlockSpec((BN, F2), lambda j: (j, 0)),
        out_shape=jax.ShapeDtypeStruct((N, F2), jnp.float32),
    )(aggp, cntp, h, wl, wr, bs, wg)


def _gcn_post_body(gp_ref, bg_ref, h_ref, out_ref):
    out_ref[...] = _leaky(gp_ref[0] + gp_ref[1] + bg_ref[...]) + h_ref[...]


def _gcn_post(gp, bg, h):
    return pl.pallas_call(
        _gcn_post_body,
        grid=(NB,),
        in_specs=[
            pl.BlockSpec((2, BN, F2), lambda j: (0, j, 0)),
            pl.BlockSpec((1, F2), lambda j: (0, 0)),
            pl.BlockSpec((BN, F2), lambda j: (j, 0)),
        ],
        out_specs=pl.BlockSpec((BN, F2), lambda j: (j, 0)),
        out_shape=jax.ShapeDtypeStruct((N, F2), jnp.float32),
    )(gp, bg, h)


def _pool_mlp_body(h_ref, b_ref, w1_ref, b1_ref, w2_ref, b2_ref, w3_ref,
                   b3_ref, w4_ref, b4_ref, out_ref, m_ref, s1_ref, s2_ref):
    p = pl.program_id(0)
    j = pl.program_id(1)
    bcol = b_ref[0]

    @pl.when(jnp.logical_and(p == 0, j == 0))
    def _():
        m_ref[...] = jnp.full((B, F2), -1e30, jnp.float32)

    @pl.when(p == 0)
    def _():
        x = h_ref[...]

        def sbody(sg, _):
            mask = bcol == sg
            mx = jnp.max(jnp.where(mask, x, -1e30), axis=0, keepdims=True)
            m_ref[pl.ds(sg, 1), :] = jnp.maximum(m_ref[pl.ds(sg, 1), :], mx)
            return 0

        lax.fori_loop(jnp.min(bcol), jnp.max(bcol) + 1, sbody, 0)

    @pl.when(p == 1)
    def _():
        @pl.when(j == 0)
        def _():
            s1_ref[...] = jnp.zeros((B, F2), jnp.float32)
            s2_ref[...] = jnp.zeros((B, F2), jnp.float32)

        x = h_ref[...]
        oh = (lax.broadcasted_iota(jnp.int32, (BN, B), 1)
              == bcol).astype(jnp.float32)
        mrow = jnp.dot(oh, m_ref[...], preferred_element_type=jnp.float32)
        e = jnp.exp(x - mrow)
        contract = (((0,), (0,)), ((), ()))
        s1_ref[...] += lax.dot_general(oh, e, contract,
                                       preferred_element_type=jnp.float32)
        s2_ref[...] += lax.dot_general(oh, e * x, contract,
                                       preferred_element_type=jnp.float32)

        @pl.when(j == NB - 1)
        def _():
            s1 = s1_ref[...]
            g = jnp.where(s1 > 0, s2_ref[...] / s1, 0.0)
            o = _leaky(_dot_t(g, w1_ref[...]) + b1_ref[...])
            o = _leaky(_dot_t(o, w2_ref[...]) + b2_ref[...])
            o = _leaky(_dot_t(o, w3_ref[...]) + b3_ref[...])
            out_ref[...] = _dot_t(o, w4_ref[...]) + b4_ref[...]


def _pool_mlp(h, batchb, w1, b1, w2, b2, w3, b3, w4, b4):
    full = lambda j_shape: None
    return pl.pallas_call(
        _pool_mlp_body,
        grid=(2, NB),
        in_specs=[
            pl.BlockSpec((BN, F2), lambda p, j: (j, 0)),
            pl.BlockSpec((1, BN, 1), lambda p, j: (j, 0, 0)),
            pl.BlockSpec((F2, F2), lambda p, j: (0, 0)),
            pl.BlockSpec((1, F2), lambda p, j: (0, 0)),
            pl.BlockSpec((64, F2), lambda p, j: (0, 0)),
            pl.BlockSpec((1, 64), lambda p, j: (0, 0)),
            pl.BlockSpec((64, 64), lambda p, j: (0, 0)),
            pl.BlockSpec((1, 64), lambda p, j: (0, 0)),
            pl.BlockSpec((A, 64), lambda p, j: (0, 0)),
            pl.BlockSpec((1, A), lambda p, j: (0, 0)),
        ],
        out_specs=pl.BlockSpec((B, A), lambda p, j: (0, 0)),
        out_shape=jax.ShapeDtypeStruct((B, A), jnp.float32),
        scratch_shapes=[
            pltpu.VMEM((B, F2), jnp.float32),
            pltpu.VMEM((B, F2), jnp.float32),
            pltpu.VMEM((B, F2), jnp.float32),
        ],
    )(h, batchb, w1, b1, w2, b2, w3, b3, w4, b4)


def kernel(x, edge_index, batch, emb, b1_Wl, b1_Wr, b1_bs, b1_Wg, b1_bg,
           b2_Wl, b2_Wr, b2_bs, b2_Wg, b2_bg, W1, bm1, W2, bm2, W3, bm3,
           W4, bm4):
    xb = x.T.astype(jnp.int32).reshape(2, NB, BN).transpose(1, 0, 2)
    pad = EPAD - E
    src2 = jnp.concatenate(
        [edge_index[0].astype(jnp.int32), jnp.zeros((pad,), jnp.int32)]
    ).reshape(NROWS, CW)
    dst2 = jnp.concatenate(
        [edge_index[1].astype(jnp.int32),
         jnp.full((pad,), NPAD - 1, jnp.int32)]
    ).reshape(NROWS, CW)
    batchb = batch.astype(jnp.int32).reshape(NB, BN, 1)
    consts = (jnp.zeros((NSEG, F2), jnp.float32),
              jnp.zeros((NSEG, CNTW), jnp.float32),
              jnp.ones((CW, CNTW), jnp.float32))
    r2 = lambda v: v.reshape(1, -1)

    h1 = _embed(xb, emb)
    cnt1 = _sc_cnt()(dst2, consts[1], consts[2])
    agg1 = _seg_sum(h1, src2, dst2, consts)
    t1 = _sage_gcn_pre(agg1, cnt1, h1, b1_Wl, b1_Wr, r2(b1_bs), b1_Wg)
    g1 = _seg_sum(t1, src2, dst2, consts)
    h2 = _gcn_post(g1, r2(b1_bg), h1)
    agg2 = _seg_sum(h2, src2, dst2, consts)
    t2 = _sage_gcn_pre(agg2, cnt1, h2, b2_Wl, b2_Wr, r2(b2_bs), b2_Wg)
    g2 = _seg_sum(t2, src2, dst2, consts)
    h3 = _gcn_post(g2, r2(b2_bg), h2)
    return _pool_mlp(h3, batchb, W1, r2(bm1), W2, r2(bm2), W3, r2(bm3),
                     W4, r2(bm4))


# speedup vs baseline: 1.0869x; 1.0869x over previous
import functools

import jax
import jax.numpy as jnp
from jax import lax
from jax.experimental import pallas as pl
from jax.experimental.pallas import tpu as pltpu
from jax.experimental.pallas import tpu_sc as plsc

N = 10000
E = 320000
F2 = 128
B = 64
A = 256

CW = 125
EPAD = E
NROWS = EPAD // CW
NWORK = 32
ROWS_W = NROWS // NWORK
NPAD = 10240
NSEG = NPAD // 16
CNTW = 128
TS = 16


def _leaky(v):
    return jnp.where(v >= 0, v, 0.01 * v)


def _dot_t(a, w):
    return lax.dot_general(a, w, (((1,), (1,)), ((), ())),
                           preferred_element_type=jnp.float32)


def _sc_segsum_body(h_hbm, src_hbm, dst_hbm, zrow_hbm, agg_hbm,
                    src_v, dst_v, b0, b1, acc_sh, g0, g1):
    c = lax.axis_index("c")
    s = lax.axis_index("s")
    base = (c * 16 + s) * ROWS_W
    bufs = (b0, b1)
    gsem = (g0, g1)

    pltpu.sync_copy(zrow_hbm, acc_sh.at[pl.ds(s * NSEG, NSEG)])
    plsc.subcore_barrier()

    def gat(q, j):
        pltpu.async_copy(h_hbm.at[src_v.at[q]], bufs[j], gsem[j])

    def wg(j):
        pltpu.make_async_copy(h_hbm.at[src_v.at[0]], bufs[j], gsem[j]).wait()

    def sca(q, j):
        pltpu.sync_copy(bufs[j], acc_sh.at[dst_v.at[q]], add=True)

    for t in range(ROWS_W // TS):
        pltpu.sync_copy(src_hbm.at[pl.ds(base + t * TS, TS)], src_v)
        pltpu.sync_copy(dst_hbm.at[pl.ds(base + t * TS, TS)], dst_v)
        gat(0, 0)

        def step(k, _):
            q0 = 2 * k
            wg(0)
            gat(q0 + 1, 1)
            sca(q0, 0)
            wg(1)
            gat((q0 + 2) % TS, 0)
            sca(q0 + 1, 1)
            return 0

        lax.fori_loop(0, TS // 2, step, 0)
        wg(0)

    plsc.subcore_barrier()
    sl = pl.ds(s * NSEG, NSEG)
    pltpu.sync_copy(acc_sh.at[sl], agg_hbm.at[c].at[sl])


@functools.cache
def _sc_segsum():
    return pl.kernel(
        _sc_segsum_body,
        out_type=jax.ShapeDtypeStruct((2, NPAD, F2), jnp.float32),
        mesh=plsc.VectorSubcoreMesh(core_axis_name="c", subcore_axis_name="s",
                                    num_cores=2, num_subcores=16),
        scratch_types=[
            pltpu.VMEM((TS, CW), jnp.int32),
            pltpu.VMEM((TS, CW), jnp.int32),
            pltpu.VMEM((CW, F2), jnp.float32),
            pltpu.VMEM((CW, F2), jnp.float32),
            pltpu.VMEM_SHARED((NPAD, F2), jnp.float32),
            pltpu.SemaphoreType.DMA,
            pltpu.SemaphoreType.DMA,
        ],
    )


def _sc_cnt_body(dst_hbm, zcnt_hbm, ones_hbm, cnt_hbm,
                 dst_v, ones_v, cnt_sh):
    c = lax.axis_index("c")
    s = lax.axis_index("s")
    base = (c * 16 + s) * ROWS_W

    pltpu.sync_copy(ones_hbm, ones_v)
    pltpu.sync_copy(zcnt_hbm, cnt_sh.at[pl.ds(s * NSEG, NSEG)])
    plsc.subcore_barrier()

    for t in range(ROWS_W // TS):
        pltpu.sync_copy(dst_hbm.at[pl.ds(base + t * TS, TS)], dst_v)

        def step(q, _):
            pltpu.sync_copy(ones_v, cnt_sh.at[dst_v.at[q]], add=True)
            return 0

        lax.fori_loop(0, TS, step, 0)

    plsc.subcore_barrier()
    sl = pl.ds(s * NSEG, NSEG)
    pltpu.sync_copy(cnt_sh.at[sl], cnt_hbm.at[c].at[sl])


@functools.cache
def _sc_cnt():
    return pl.kernel(
        _sc_cnt_body,
        out_type=jax.ShapeDtypeStruct((2, NPAD, CNTW), jnp.float32),
        mesh=plsc.VectorSubcoreMesh(core_axis_name="c", subcore_axis_name="s",
                                    num_cores=2, num_subcores=16),
        scratch_types=[
            pltpu.VMEM((TS, CW), jnp.int32),
            pltpu.VMEM((CW, CNTW), jnp.float32),
            pltpu.VMEM_SHARED((NPAD, CNTW), jnp.float32),
        ],
    )


def _seg_sum(h, src2, dst2, consts):
    return _sc_segsum()(h, src2, dst2, consts[0])


BN = 1000
NB = N // BN


def _embed_body(x_ref, emb_ref, out_ref):
    bn = out_ref.shape[0]
    lanes = lax.broadcasted_iota(jnp.int32, (bn, F2), 1)
    oh0 = (lanes == x_ref[0, 0, :][:, None]).astype(jnp.float32)
    oh1 = (lanes == x_ref[0, 1, :][:, None]).astype(jnp.float32)
    h0 = jnp.dot(oh0, emb_ref[...], preferred_element_type=jnp.float32)
    h1 = jnp.dot(oh1, emb_ref[...], preferred_element_type=jnp.float32)
    out_ref[...] = jnp.concatenate([h0, h1], axis=1)


def _embed(xb, emb):
    return pl.pallas_call(
        _embed_body,
        grid=(NB,),
        in_specs=[
            pl.BlockSpec((1, 2, BN), lambda j: (j, 0, 0)),
            pl.BlockSpec((F2, 64), lambda j: (0, 0)),
        ],
        out_specs=pl.BlockSpec((BN, F2), lambda j: (j, 0)),
        out_shape=jax.ShapeDtypeStruct((N, F2), jnp.float32),
    )(xb, emb)


def _sage_body(aggp_ref, cntp_ref, h_ref, wl_ref, wr_ref, bs_ref, wg_ref,
               out_ref):
    agg = aggp_ref[0] + aggp_ref[1]
    cnt = cntp_ref[0, :, 0] + cntp_ref[1, :, 0]
    mean = agg / jnp.maximum(cnt, 1.0)[:, None]
    h = h_ref[...]
    sa = _leaky(_dot_t(mean, wl_ref[...]) + _dot_t(h, wr_ref[...])
                + bs_ref[...])
    out_ref[...] = _dot_t(sa, wg_ref[...])


def _sage_gcn_pre(aggp, cntp, h, wl, wr, bs, wg):
    return pl.pallas_call(
        _sage_body,
        grid=(NB,),
        in_specs=[
            pl.BlockSpec((2, BN, F2), lambda j: (0, j, 0)),
            pl.BlockSpec((2, BN, CNTW), lambda j: (0, j, 0)),
            pl.BlockSpec((BN, F2), lambda j: (j, 0)),
            pl.BlockSpec((F2, F2), lambda j: (0, 0)),
            pl.BlockSpec((F2, F2), lambda j: (0, 0)),
            pl.BlockSpec((1, F2), lambda j: (0, 0)),
            pl.BlockSpec((F2, F2), lambda j: (0, 0)),
        ],
        out_specs=pl.BlockSpec((BN, F2), lambda j: (j, 0)),
        out_shape=jax.ShapeDtypeStruct((N, F2), jnp.float32),
    )(aggp, cntp, h, wl, wr, bs, wg)


def _gcn_post_body(gp_ref, bg_ref, h_ref, out_ref):
    out_ref[...] = _leaky(gp_ref[0] + gp_ref[1] + bg_ref[...]) + h_ref[...]


def _gcn_post(gp, bg, h):
    return pl.pallas_call(
        _gcn_post_body,
        grid=(NB,),
        in_specs=[
            pl.BlockSpec((2, BN, F2), lambda j: (0, j, 0)),
            pl.BlockSpec((1, F2), lambda j: (0, 0)),
            pl.BlockSpec((BN, F2), lambda j: (j, 0)),
        ],
        out_specs=pl.BlockSpec((BN, F2), lambda j: (j, 0)),
        out_shape=jax.ShapeDtypeStruct((N, F2), jnp.float32),
    )(gp, bg, h)


def _pool_mlp_body(h_ref, b_ref, w1_ref, b1_ref, w2_ref, b2_ref, w3_ref,
                   b3_ref, w4_ref, b4_ref, out_ref, m_ref, s1_ref, s2_ref):
    p = pl.program_id(0)
    j = pl.program_id(1)
    bcol = b_ref[0]

    @pl.when(jnp.logical_and(p == 0, j == 0))
    def _():
        m_ref[...] = jnp.full((B, F2), -1e30, jnp.float32)

    @pl.when(p == 0)
    def _():
        x = h_ref[...]

        def sbody(sg, _):
            mask = bcol == sg
            mx = jnp.max(jnp.where(mask, x, -1e30), axis=0, keepdims=True)
            m_ref[pl.ds(sg, 1), :] = jnp.maximum(m_ref[pl.ds(sg, 1), :], mx)
            return 0

        lax.fori_loop(jnp.min(bcol), jnp.max(bcol) + 1, sbody, 0)

    @pl.when(p == 1)
    def _():
        @pl.when(j == 0)
        def _():
            s1_ref[...] = jnp.zeros((B, F2), jnp.float32)
            s2_ref[...] = jnp.zeros((B, F2), jnp.float32)

        x = h_ref[...]
        oh = (lax.broadcasted_iota(jnp.int32, (BN, B), 1)
              == bcol).astype(jnp.float32)
        mrow = jnp.dot(oh, m_ref[...], preferred_element_type=jnp.float32)
        e = jnp.exp(x - mrow)
        contract = (((0,), (0,)), ((), ()))
        s1_ref[...] += lax.dot_general(oh, e, contract,
                                       preferred_element_type=jnp.float32)
        s2_ref[...] += lax.dot_general(oh, e * x, contract,
                                       preferred_element_type=jnp.float32)

        @pl.when(j == NB - 1)
        def _():
            s1 = s1_ref[...]
            g = jnp.where(s1 > 0, s2_ref[...] / s1, 0.0)
            o = _leaky(_dot_t(g, w1_ref[...]) + b1_ref[...])
            o = _leaky(_dot_t(o, w2_ref[...]) + b2_ref[...])
            o = _leaky(_dot_t(o, w3_ref[...]) + b3_ref[...])
            out_ref[...] = _dot_t(o, w4_ref[...]) + b4_ref[...]


def _pool_mlp(h, batchb, w1, b1, w2, b2, w3, b3, w4, b4):
    full = lambda j_shape: None
    return pl.pallas_call(
        _pool_mlp_body,
        grid=(2, NB),
        in_specs=[
            pl.BlockSpec((BN, F2), lambda p, j: (j, 0)),
            pl.BlockSpec((1, BN, 1), lambda p, j: (j, 0, 0)),
            pl.BlockSpec((F2, F2), lambda p, j: (0, 0)),
            pl.BlockSpec((1, F2), lambda p, j: (0, 0)),
            pl.BlockSpec((64, F2), lambda p, j: (0, 0)),
            pl.BlockSpec((1, 64), lambda p, j: (0, 0)),
            pl.BlockSpec((64, 64), lambda p, j: (0, 0)),
            pl.BlockSpec((1, 64), lambda p, j: (0, 0)),
            pl.BlockSpec((A, 64), lambda p, j: (0, 0)),
            pl.BlockSpec((1, A), lambda p, j: (0, 0)),
        ],
        out_specs=pl.BlockSpec((B, A), lambda p, j: (0, 0)),
        out_shape=jax.ShapeDtypeStruct((B, A), jnp.float32),
        scratch_shapes=[
            pltpu.VMEM((B, F2), jnp.float32),
            pltpu.VMEM((B, F2), jnp.float32),
            pltpu.VMEM((B, F2), jnp.float32),
        ],
    )(h, batchb, w1, b1, w2, b2, w3, b3, w4, b4)


def kernel(x, edge_index, batch, emb, b1_Wl, b1_Wr, b1_bs, b1_Wg, b1_bg,
           b2_Wl, b2_Wr, b2_bs, b2_Wg, b2_bg, W1, bm1, W2, bm2, W3, bm3,
           W4, bm4):
    xb = x.T.astype(jnp.int32).reshape(2, NB, BN).transpose(1, 0, 2)
    pad = EPAD - E
    src2 = jnp.concatenate(
        [edge_index[0].astype(jnp.int32), jnp.zeros((pad,), jnp.int32)]
    ).reshape(NROWS, CW)
    dst2 = jnp.concatenate(
        [edge_index[1].astype(jnp.int32),
         jnp.full((pad,), NPAD - 1, jnp.int32)]
    ).reshape(NROWS, CW)
    batchb = batch.astype(jnp.int32).reshape(NB, BN, 1)
    consts = (jnp.zeros((NSEG, F2), jnp.float32),
              jnp.zeros((NSEG, CNTW), jnp.float32),
              jnp.ones((CW, CNTW), jnp.float32))
    r2 = lambda v: v.reshape(1, -1)

    h1 = _embed(xb, emb)
    cnt1 = _sc_cnt()(dst2, consts[1], consts[2])
    agg1 = _seg_sum(h1, src2, dst2, consts)
    t1 = _sage_gcn_pre(agg1, cnt1, h1, b1_Wl, b1_Wr, r2(b1_bs), b1_Wg)
    g1 = _seg_sum(t1, src2, dst2, consts)
    h2 = _gcn_post(g1, r2(b1_bg), h1)
    agg2 = _seg_sum(h2, src2, dst2, consts)
    t2 = _sage_gcn_pre(agg2, cnt1, h2, b2_Wl, b2_Wr, r2(b2_bs), b2_Wg)
    g2 = _seg_sum(t2, src2, dst2, consts)
    h3 = _gcn_post(g2, r2(b2_bg), h2)
    return _pool_mlp(h3, batchb, W1, r2(bm1), W2, r2(bm2), W3, r2(bm3),
                     W4, r2(bm4))


# speedup vs baseline: 1.0944x; 1.0069x over previous
import functools

import jax
import jax.numpy as jnp
from jax import lax
from jax.experimental import pallas as pl
from jax.experimental.pallas import tpu as pltpu
from jax.experimental.pallas import tpu_sc as plsc

N = 10000
E = 320000
F2 = 128
B = 64
A = 256

CW = 125
EPAD = E
NROWS = EPAD // CW
NWORK = 32
ROWS_W = NROWS // NWORK
NPAD = 10240
NSEG = NPAD // 16
CNTW = 128
TS = 16


def _leaky(v):
    return jnp.where(v >= 0, v, 0.01 * v)


def _dot_t(a, w):
    return lax.dot_general(a, w, (((1,), (1,)), ((), ())),
                           preferred_element_type=jnp.float32)


def _sc_segsum_body(h_hbm, src_hbm, dst_hbm, zrow_hbm, agg_hbm,
                    src_v, dst_v, b0, b1, acc_sh, g0, g1):
    c = lax.axis_index("c")
    s = lax.axis_index("s")
    base = (c * 16 + s) * ROWS_W
    bufs = (b0, b1)
    gsem = (g0, g1)

    pltpu.sync_copy(zrow_hbm, acc_sh.at[pl.ds(s * NSEG, NSEG)])
    plsc.subcore_barrier()

    def gat(q, j):
        pltpu.async_copy(h_hbm.at[src_v.at[q]], bufs[j], gsem[j])

    def wg(j):
        pltpu.make_async_copy(h_hbm.at[src_v.at[0]], bufs[j], gsem[j]).wait()

    def sca(q, j):
        pltpu.sync_copy(bufs[j], acc_sh.at[dst_v.at[q]], add=True)

    for t in range(ROWS_W // TS):
        pltpu.sync_copy(src_hbm.at[pl.ds(base + t * TS, TS)], src_v)
        pltpu.sync_copy(dst_hbm.at[pl.ds(base + t * TS, TS)], dst_v)
        gat(0, 0)

        def step(k, _):
            q0 = 2 * k
            wg(0)
            gat(q0 + 1, 1)
            sca(q0, 0)
            wg(1)
            gat((q0 + 2) % TS, 0)
            sca(q0 + 1, 1)
            return 0

        lax.fori_loop(0, TS // 2, step, 0)
        wg(0)

    plsc.subcore_barrier()
    sl = pl.ds(s * NSEG, NSEG)
    pltpu.sync_copy(acc_sh.at[sl], agg_hbm.at[c].at[sl])


@functools.cache
def _sc_segsum():
    return pl.kernel(
        _sc_segsum_body,
        out_type=jax.ShapeDtypeStruct((2, NPAD, F2), jnp.float32),
        mesh=plsc.VectorSubcoreMesh(core_axis_name="c", subcore_axis_name="s",
                                    num_cores=2, num_subcores=16),
        scratch_types=[
            pltpu.VMEM((TS, CW), jnp.int32),
            pltpu.VMEM((TS, CW), jnp.int32),
            pltpu.VMEM((CW, F2), jnp.float32),
            pltpu.VMEM((CW, F2), jnp.float32),
            pltpu.VMEM_SHARED((NPAD, F2), jnp.float32),
            pltpu.SemaphoreType.DMA,
            pltpu.SemaphoreType.DMA,
        ],
    )


def _sc_cnt_body(dst_hbm, zcnt_hbm, ones_hbm, cnt_hbm,
                 dst_v, ones_v, cnt_sh):
    c = lax.axis_index("c")
    s = lax.axis_index("s")
    base = (c * 16 + s) * ROWS_W

    pltpu.sync_copy(ones_hbm, ones_v)
    pltpu.sync_copy(zcnt_hbm, cnt_sh.at[pl.ds(s * NSEG, NSEG)])
    plsc.subcore_barrier()

    for t in range(ROWS_W // TS):
        pltpu.sync_copy(dst_hbm.at[pl.ds(base + t * TS, TS)], dst_v)

        def step(q, _):
            pltpu.sync_copy(ones_v, cnt_sh.at[dst_v.at[q]], add=True)
            return 0

        lax.fori_loop(0, TS, step, 0)

    plsc.subcore_barrier()
    sl = pl.ds(s * NSEG, NSEG)
    pltpu.sync_copy(cnt_sh.at[sl], cnt_hbm.at[c].at[sl])


@functools.cache
def _sc_cnt():
    return pl.kernel(
        _sc_cnt_body,
        out_type=jax.ShapeDtypeStruct((2, NPAD, CNTW), jnp.float32),
        mesh=plsc.VectorSubcoreMesh(core_axis_name="c", subcore_axis_name="s",
                                    num_cores=2, num_subcores=16),
        scratch_types=[
            pltpu.VMEM((TS, CW), jnp.int32),
            pltpu.VMEM((CW, CNTW), jnp.float32),
            pltpu.VMEM_SHARED((NPAD, CNTW), jnp.float32),
        ],
    )


def _seg_sum(h, src2, dst2, consts):
    return _sc_segsum()(h, src2, dst2, consts[0])


BN = 1000
NB = N // BN


def _embed_body(x_ref, emb_ref, out_ref):
    bn = out_ref.shape[0]
    lanes = lax.broadcasted_iota(jnp.int32, (bn, F2), 1)
    oh0 = (lanes == x_ref[0, 0, :][:, None]).astype(jnp.float32)
    oh1 = (lanes == x_ref[0, 1, :][:, None]).astype(jnp.float32)
    h0 = jnp.dot(oh0, emb_ref[...], preferred_element_type=jnp.float32)
    h1 = jnp.dot(oh1, emb_ref[...], preferred_element_type=jnp.float32)
    out_ref[...] = jnp.concatenate([h0, h1], axis=1)


def _embed(xb, emb):
    return pl.pallas_call(
        _embed_body,
        grid=(NB,),
        in_specs=[
            pl.BlockSpec((1, 2, BN), lambda j: (j, 0, 0)),
            pl.BlockSpec((F2, 64), lambda j: (0, 0)),
        ],
        out_specs=pl.BlockSpec((BN, F2), lambda j: (j, 0)),
        out_shape=jax.ShapeDtypeStruct((N, F2), jnp.float32),
    )(xb, emb)


def _sage_body(aggp_ref, cntp_ref, h_ref, wl_ref, wr_ref, bs_ref, wg_ref,
               out_ref):
    agg = aggp_ref[0] + aggp_ref[1]
    cnt = cntp_ref[0, :, 0] + cntp_ref[1, :, 0]
    mean = agg / jnp.maximum(cnt, 1.0)[:, None]
    h = h_ref[...]
    sa = _leaky(_dot_t(mean, wl_ref[...]) + _dot_t(h, wr_ref[...])
                + bs_ref[...])
    out_ref[...] = _dot_t(sa, wg_ref[...])


def _sage_gcn_pre(aggp, cntp, h, wl, wr, bs, wg):
    return pl.pallas_call(
        _sage_body,
        grid=(NB,),
        in_specs=[
            pl.BlockSpec((2, BN, F2), lambda j: (0, j, 0)),
            pl.BlockSpec((2, BN, CNTW), lambda j: (0, j, 0)),
            pl.BlockSpec((BN, F2), lambda j: (j, 0)),
            pl.BlockSpec((F2, F2), lambda j: (0, 0)),
            pl.BlockSpec((F2, F2), lambda j: (0, 0)),
            pl.BlockSpec((1, F2), lambda j: (0, 0)),
            pl.BlockSpec((F2, F2), lambda j: (0, 0)),
        ],
        out_specs=pl.BlockSpec((BN, F2), lambda j: (j, 0)),
        out_shape=jax.ShapeDtypeStruct((N, F2), jnp.float32),
    )(aggp, cntp, h, wl, wr, bs, wg)


def _gcn_post_body(gp_ref, bg_ref, h_ref, out_ref):
    out_ref[...] = _leaky(gp_ref[0] + gp_ref[1] + bg_ref[...]) + h_ref[...]


def _gcn_post(gp, bg, h):
    return pl.pallas_call(
        _gcn_post_body,
        grid=(NB,),
        in_specs=[
            pl.BlockSpec((2, BN, F2), lambda j: (0, j, 0)),
            pl.BlockSpec((1, F2), lambda j: (0, 0)),
            pl.BlockSpec((BN, F2), lambda j: (j, 0)),
        ],
        out_specs=pl.BlockSpec((BN, F2), lambda j: (j, 0)),
        out_shape=jax.ShapeDtypeStruct((N, F2), jnp.float32),
    )(gp, bg, h)


def _pool_mlp_body(gp_ref, bg_ref, h_ref, b_ref, w1_ref, b1_ref, w2_ref,
                   b2_ref, w3_ref, b3_ref, w4_ref, b4_ref, out_ref,
                   m_ref, s1_ref, s2_ref):
    p = pl.program_id(0)
    j = pl.program_id(1)
    bcol = b_ref[0]

    def final_h():
        return (_leaky(gp_ref[0] + gp_ref[1] + bg_ref[...]) + h_ref[...])

    @pl.when(jnp.logical_and(p == 0, j == 0))
    def _():
        m_ref[...] = jnp.full((B, F2), -1e30, jnp.float32)

    @pl.when(p == 0)
    def _():
        x = final_h()

        def sbody(sg, _):
            mask = bcol == sg
            mx = jnp.max(jnp.where(mask, x, -1e30), axis=0, keepdims=True)
            m_ref[pl.ds(sg, 1), :] = jnp.maximum(m_ref[pl.ds(sg, 1), :], mx)
            return 0

        lax.fori_loop(jnp.min(bcol), jnp.max(bcol) + 1, sbody, 0)

    @pl.when(p == 1)
    def _():
        @pl.when(j == 0)
        def _():
            s1_ref[...] = jnp.zeros((B, F2), jnp.float32)
            s2_ref[...] = jnp.zeros((B, F2), jnp.float32)

        x = final_h()
        oh = (lax.broadcasted_iota(jnp.int32, (BN, B), 1)
              == bcol).astype(jnp.float32)
        mrow = jnp.dot(oh, m_ref[...], preferred_element_type=jnp.float32)
        e = jnp.exp(x - mrow)
        contract = (((0,), (0,)), ((), ()))
        s1_ref[...] += lax.dot_general(oh, e, contract,
                                       preferred_element_type=jnp.float32)
        s2_ref[...] += lax.dot_general(oh, e * x, contract,
                                       preferred_element_type=jnp.float32)

        @pl.when(j == NB - 1)
        def _():
            s1 = s1_ref[...]
            g = jnp.where(s1 > 0, s2_ref[...] / s1, 0.0)
            o = _leaky(_dot_t(g, w1_ref[...]) + b1_ref[...])
            o = _leaky(_dot_t(o, w2_ref[...]) + b2_ref[...])
            o = _leaky(_dot_t(o, w3_ref[...]) + b3_ref[...])
            out_ref[...] = _dot_t(o, w4_ref[...]) + b4_ref[...]


def _pool_mlp(gp, bg, h, batchb, w1, b1, w2, b2, w3, b3, w4, b4):
    return pl.pallas_call(
        _pool_mlp_body,
        grid=(2, NB),
        in_specs=[
            pl.BlockSpec((2, BN, F2), lambda p, j: (0, j, 0)),
            pl.BlockSpec((1, F2), lambda p, j: (0, 0)),
            pl.BlockSpec((BN, F2), lambda p, j: (j, 0)),
            pl.BlockSpec((1, BN, 1), lambda p, j: (j, 0, 0)),
            pl.BlockSpec((F2, F2), lambda p, j: (0, 0)),
            pl.BlockSpec((1, F2), lambda p, j: (0, 0)),
            pl.BlockSpec((64, F2), lambda p, j: (0, 0)),
            pl.BlockSpec((1, 64), lambda p, j: (0, 0)),
            pl.BlockSpec((64, 64), lambda p, j: (0, 0)),
            pl.BlockSpec((1, 64), lambda p, j: (0, 0)),
            pl.BlockSpec((A, 64), lambda p, j: (0, 0)),
            pl.BlockSpec((1, A), lambda p, j: (0, 0)),
        ],
        out_specs=pl.BlockSpec((B, A), lambda p, j: (0, 0)),
        out_shape=jax.ShapeDtypeStruct((B, A), jnp.float32),
        scratch_shapes=[
            pltpu.VMEM((B, F2), jnp.float32),
            pltpu.VMEM((B, F2), jnp.float32),
            pltpu.VMEM((B, F2), jnp.float32),
        ],
    )(gp, bg, h, batchb, w1, b1, w2, b2, w3, b3, w4, b4)


def kernel(x, edge_index, batch, emb, b1_Wl, b1_Wr, b1_bs, b1_Wg, b1_bg,
           b2_Wl, b2_Wr, b2_bs, b2_Wg, b2_bg, W1, bm1, W2, bm2, W3, bm3,
           W4, bm4):
    xb = x.T.astype(jnp.int32).reshape(2, NB, BN).transpose(1, 0, 2)
    pad = EPAD - E
    src2 = jnp.concatenate(
        [edge_index[0].astype(jnp.int32), jnp.zeros((pad,), jnp.int32)]
    ).reshape(NROWS, CW)
    dst2 = jnp.concatenate(
        [edge_index[1].astype(jnp.int32),
         jnp.full((pad,), NPAD - 1, jnp.int32)]
    ).reshape(NROWS, CW)
    batchb = batch.astype(jnp.int32).reshape(NB, BN, 1)
    consts = (jnp.zeros((NSEG, F2), jnp.float32),
              jnp.zeros((NSEG, CNTW), jnp.float32),
              jnp.ones((CW, CNTW), jnp.float32))
    r2 = lambda v: v.reshape(1, -1)

    h1 = _embed(xb, emb)
    cnt1 = _sc_cnt()(dst2, consts[1], consts[2])
    agg1 = _seg_sum(h1, src2, dst2, consts)
    t1 = _sage_gcn_pre(agg1, cnt1, h1, b1_Wl, b1_Wr, r2(b1_bs), b1_Wg)
    g1 = _seg_sum(t1, src2, dst2, consts)
    h2 = _gcn_post(g1, r2(b1_bg), h1)
    agg2 = _seg_sum(h2, src2, dst2, consts)
    t2 = _sage_gcn_pre(agg2, cnt1, h2, b2_Wl, b2_Wr, r2(b2_bs), b2_Wg)
    g2 = _seg_sum(t2, src2, dst2, consts)
    return _pool_mlp(g2, r2(b2_bg), h2, batchb, W1, r2(bm1), W2, r2(bm2),
                     W3, r2(bm3), W4, r2(bm4))
